# Initial kernel scaffold; baseline (speedup 1.0000x reference)
import jax
import jax.numpy as jnp
from jax.experimental import pallas as pl


def kernel(boxes, labels):
    raise NotImplementedError("write your pallas kernel here")


# speedup vs baseline: 9.8176x; 9.8176x over previous
import math

import jax
import jax.numpy as jnp
from jax.experimental import pallas as pl

_STRIDES = (8, 16, 32, 64, 128)
_IMG = 512
_NCLS = 21
_NB = 64


def _thresholds():
    result = []
    last = _IMG
    for i in range(len(_STRIDES) - 1, -1, -1):
        s = _STRIDES[i]
        px = float(s) / _IMG
        th_max = math.ceil(last / s)
        if th_max % 2:
            th_max += 1
        th_min = th_max // 2
        last = th_min * s
        if i == 0:
            th_min = 1
        result.append((th_min * px, th_max * px))
    return tuple(result[::-1])

_THS = _thresholds()


def _mapper_kernel(boxes_ref, boxesT_ref, labels_ref, *out_refs):
    bx = boxes_ref[0]
    bT = boxesT_ref[0]
    lab = labels_ref[0]

    x1c = bx[:, 0:1].reshape(_NB, 1, 1)
    y1c = bx[:, 1:2].reshape(_NB, 1, 1)
    x2c = bx[:, 2:3].reshape(_NB, 1, 1)
    y2c = bx[:, 3:4].reshape(_NB, 1, 1)
    labf = lab.astype(jnp.float32).reshape(_NB, 1, 1)

    area_col = (bx[:, 2:3] - bx[:, 0:1]) * (bx[:, 3:4] - bx[:, 1:2])
    area_row = (bT[2:3, :] - bT[0:1, :]) * (bT[3:4, :] - bT[1:2, :])
    ii = jax.lax.broadcasted_iota(jnp.int32, (_NB, _NB), 0)
    jj = jax.lax.broadcasted_iota(jnp.int32, (_NB, _NB), 1)
    before = (area_row > area_col) | ((area_row == area_col) & (jj < ii))
    rank = jnp.sum(before.astype(jnp.float32), axis=1, keepdims=True)
    rankp1 = (rank + 1.0).reshape(_NB, 1, 1)

    for lev, out_ref in enumerate(out_refs):
        s = _STRIDES[lev]
        size = _IMG // s
        th0, th1 = _THS[lev]
        scale = float(s) / _IMG

        cx3 = (jax.lax.broadcasted_iota(jnp.int32, (1, size, size), 2).astype(jnp.float32)
               + 0.5) * scale
        cy3 = (jax.lax.broadcasted_iota(jnp.int32, (1, size, size), 1).astype(jnp.float32)
               + 0.5) * scale

        l3 = cx3 - x1c
        t3 = cy3 - y1c
        r3 = x2c - cx3
        b3 = y2c - cy3
        mn = jnp.minimum(jnp.minimum(l3, r3), jnp.minimum(t3, b3))
        mx = jnp.maximum(jnp.maximum(l3, r3), jnp.maximum(t3, b3))
        fg3 = mn >= 0.0
        pred3 = fg3 & (mx > th0) & (mx <= th1)

        anyfg = jnp.any(fg3, axis=0)
        wsel = jnp.where(pred3, rankp1, 0.0)
        wmax = jnp.max(wsel, axis=0)
        iwf = (pred3 & (rankp1 == wmax[None, :, :])).astype(jnp.float32)

        x1w = jnp.sum(iwf * x1c, axis=0)
        y1w = jnp.sum(iwf * y1c, axis=0)
        x2w = jnp.sum(iwf * x2c, axis=0)
        y2w = jnp.sum(iwf * y2c, axis=0)
        labw = jnp.sum(iwf * labf, axis=0)

        pf = (wmax > 0.0).astype(jnp.float32)
        cx2 = cx3[0]
        cy2 = cy3[0]
        pcx = pf * cx2
        pcy = pf * cy2
        l2 = pcx - x1w
        t2 = pcy - y1w
        r2 = x2w - pcx
        b2 = y2w - pcy

        arg = (jnp.minimum(l2, r2) / jnp.maximum(l2, r2)) * (
            jnp.minimum(t2, b2) / jnp.maximum(t2, b2))
        pos = arg > 0.0
        safe = jnp.where(pos, arg, 1.0)
        cen2 = jnp.where(pos, jnp.sqrt(safe), 0.0)

        out_ref[0, 0] = l2
        out_ref[0, 1] = t2
        out_ref[0, 2] = r2
        out_ref[0, 3] = b2
        out_ref[0, 4] = cen2
        out_ref[0, 5] = 1.0 - anyfg.astype(jnp.float32)
        for ch in range(1, _NCLS):
            out_ref[0, 5 + ch] = (labw == float(ch)).astype(jnp.float32)


def kernel(boxes, labels):
    B = boxes.shape[0]
    boxesT = jnp.swapaxes(boxes, 1, 2)
    labels3 = labels.reshape(B, _NB, 1)

    out_shapes = tuple(
        jax.ShapeDtypeStruct((B, 4 + 1 + _NCLS, _IMG // s, _IMG // s), jnp.float32)
        for s in _STRIDES)

    grid = (B,)
    in_specs = [
        pl.BlockSpec((1, _NB, 4), lambda b: (b, 0, 0)),
        pl.BlockSpec((1, 4, _NB), lambda b: (b, 0, 0)),
        pl.BlockSpec((1, _NB, 1), lambda b: (b, 0, 0)),
    ]
    out_specs = tuple(
        pl.BlockSpec((1, 4 + 1 + _NCLS, _IMG // s, _IMG // s),
                     lambda b: (b, 0, 0, 0))
        for s in _STRIDES)

    return pl.pallas_call(
        _mapper_kernel,
        grid=grid,
        in_specs=in_specs,
        out_specs=out_specs,
        out_shape=out_shapes,
    )(boxes, boxesT, labels3)


# speedup vs baseline: 10.7867x; 1.0987x over previous
import functools
import math

import jax
import jax.numpy as jnp
from jax import lax
from jax.experimental import pallas as pl
from jax.experimental.pallas import tpu as pltpu
from jax.experimental.pallas import tpu_sc as plsc

_STRIDES = (8, 16, 32, 64, 128)
_IMG = 512
_NCLS = 21
_NB = 64
_B = 8
_NC = 2
_NS = 16
_NW = _NC * _NS
_LANES = 16


def _thresholds():
    result = []
    last = _IMG
    for i in range(len(_STRIDES) - 1, -1, -1):
        s = _STRIDES[i]
        px = float(s) / _IMG
        th_max = math.ceil(last / s)
        if th_max % 2:
            th_max += 1
        th_min = th_max // 2
        last = th_min * s
        if i == 0:
            th_min = 1
        result.append((th_min * px, th_max * px))
    return tuple(result[::-1])

_THS = _thresholds()


def _sqrt_nr(a):
    i = lax.bitcast_convert_type(a, jnp.int32)
    i = jnp.int32(0x5F3759DF) - lax.shift_right_logical(i, 1)
    r = lax.bitcast_convert_type(i, jnp.float32)
    for _ in range(3):
        r = r * (1.5 - 0.5 * a * r * r)
    return a * r


def _sc_mapper(params_hbm, o0, o1, o2, o3, o4,
               params_v, areas_v, stage0, stage1, stage2, stage3, stage4):
    outs = (o0, o1, o2, o3, o4)
    stages = (stage0, stage1, stage2, stage3, stage4)
    wid = lax.axis_index("s") * _NC + lax.axis_index("c")

    pltpu.sync_copy(params_hbm, params_v)

    iota = lax.iota(jnp.int32, _LANES)
    ones = jnp.ones((_LANES,), jnp.float32)
    zeros = jnp.zeros((_LANES,), jnp.float32)

    def do_chunk(img, base, lev, col):
        s = _STRIDES[lev]
        size = _IMG // s
        log2s = size.bit_length() - 1
        th0, th1 = _THS[lev]
        scale = float(s) / _IMG
        stage = stages[lev]

        p = base + iota
        px = jnp.bitwise_and(p, size - 1)
        py = lax.shift_right_logical(p, log2s)
        cx = (px.astype(jnp.float32) + 0.5) * scale
        cy = (py.astype(jnp.float32) + 0.5) * scale

        pbase = jnp.full((_LANES,), img * 5 * _NB, jnp.int32)

        def body(i, carry):
            anyfg, warea, widxf = carry
            iv = pbase + i
            x1 = plsc.load_gather(params_v, [iv])
            y1 = plsc.load_gather(params_v, [iv + _NB])
            x2 = plsc.load_gather(params_v, [iv + 2 * _NB])
            y2 = plsc.load_gather(params_v, [iv + 3 * _NB])
            ar = plsc.load_gather(areas_v, [jnp.full((_LANES,), i, jnp.int32)])
            l = cx - x1
            t = cy - y1
            r = x2 - cx
            b = y2 - cy
            mn = jnp.minimum(jnp.minimum(l, r), jnp.minimum(t, b))
            mx = jnp.maximum(jnp.maximum(l, r), jnp.maximum(t, b))
            fg = mn >= 0.0
            pred = fg & (mx > th0) & (mx <= th1)
            anyfg = jnp.where(fg, 1.0, anyfg)
            better = pred & (ar <= warea)
            warea = jnp.where(better, ar, warea)
            widxf = jnp.where(better, i.astype(jnp.float32), widxf)
            return anyfg, warea, widxf

        init = (zeros, jnp.full((_LANES,), 3.0e38, jnp.float32),
                jnp.full((_LANES,), -1.0, jnp.float32))
        anyfg, _, widxf = lax.fori_loop(0, _NB, body, init)

        haswin = widxf >= 0.0
        wi = jnp.where(haswin, widxf, 0.0).astype(jnp.int32) + pbase

        def g(k):
            return plsc.load_gather(params_v, [wi + k * _NB])

        x1w, y1w, x2w, y2w, labw = g(0), g(1), g(2), g(3), g(4)
        lr = cx - x1w
        tr = cy - y1w
        rr = x2w - cx
        br = y2w - cy
        arg = ((jnp.minimum(lr, rr) / jnp.maximum(lr, rr)) *
               (jnp.minimum(tr, br) / jnp.maximum(tr, br)))
        pos = haswin & (arg > 0.0)
        safe = jnp.where(pos, arg, 1.0)
        cen = jnp.where(pos, _sqrt_nr(safe), 0.0)
        labm = jnp.where(haswin, labw, 0.0)

        d = pl.ds(col, _LANES)
        stage[0, d] = jnp.where(haswin, lr, 0.0)
        stage[1, d] = jnp.where(haswin, tr, 0.0)
        stage[2, d] = jnp.where(haswin, rr, 0.0)
        stage[3, d] = jnp.where(haswin, br, 0.0)
        stage[4, d] = cen
        stage[5, d] = 1.0 - anyfg
        for ch in range(1, _NCLS):
            stage[5 + ch, d] = jnp.where(labm == float(ch), 1.0, 0.0)

    def img_body(img, _):
        for k in range(_NB // _LANES):
            off = img * 5 * _NB + k * _LANES
            x1v = params_v[pl.ds(off, _LANES)]
            y1v = params_v[pl.ds(off + _NB, _LANES)]
            x2v = params_v[pl.ds(off + 2 * _NB, _LANES)]
            y2v = params_v[pl.ds(off + 3 * _NB, _LANES)]
            areas_v[pl.ds(k * _LANES, _LANES)] = (x2v - x1v) * (y2v - y1v)

        r = jnp.remainder(wid + 21 * img, 32)

        def c0(c, carry):
            do_chunk(img, r * 128 + c * _LANES, 0, c * _LANES)
            return carry
        lax.fori_loop(0, 8, c0, 0)
        pltpu.sync_copy(stage0, o0.at[img, :, pl.ds(r * 128, 128)])

        @pl.when(r < 8)
        def _():
            def c1(c, carry):
                do_chunk(img, r * 128 + c * _LANES, 1, c * _LANES)
                return carry
            lax.fori_loop(0, 8, c1, 0)
            pltpu.sync_copy(stage1, o1.at[img, :, pl.ds(r * 128, 128)])

        @pl.when((r == 8) | (r == 9))
        def _():
            k = r - 8
            def c2(c, carry):
                do_chunk(img, k * 128 + c * _LANES, 2, c * _LANES)
                return carry
            lax.fori_loop(0, 8, c2, 0)
            pltpu.sync_copy(stage2, o2.at[img, :, pl.ds(k * 128, 128)])

        @pl.when(r == 10)
        def _():
            def c3(c, carry):
                do_chunk(img, c * _LANES, 3, c * _LANES)
                return carry
            lax.fori_loop(0, 4, c3, 0)
            pltpu.sync_copy(stage3, o3.at[img, :, pl.ds(0, 64)])
            do_chunk(img, 0, 4, 0)
            pltpu.sync_copy(stage4, o4.at[img, :, pl.ds(0, 16)])

        return 0

    lax.fori_loop(0, _B, img_body, 0)


def kernel(boxes, labels):
    params = jnp.concatenate(
        [jnp.swapaxes(boxes, 1, 2), labels[:, None, :].astype(jnp.float32)],
        axis=1).reshape(-1)

    nch = 4 + 1 + _NCLS
    out_type = tuple(
        jax.ShapeDtypeStruct((_B, nch, (_IMG // s) * (_IMG // s)), jnp.float32)
        for s in _STRIDES)

    mesh = plsc.VectorSubcoreMesh(core_axis_name="c", subcore_axis_name="s",
                                  num_cores=_NC, num_subcores=_NS)
    run = pl.kernel(
        _sc_mapper,
        out_type=out_type,
        mesh=mesh,
        compiler_params=pltpu.CompilerParams(needs_layout_passes=False),
        scratch_types=[
            pltpu.VMEM((_B * 5 * _NB,), jnp.float32),
            pltpu.VMEM((_NB,), jnp.float32),
            pltpu.VMEM((nch, 128), jnp.float32),
            pltpu.VMEM((nch, 128), jnp.float32),
            pltpu.VMEM((nch, 128), jnp.float32),
            pltpu.VMEM((nch, 64), jnp.float32),
            pltpu.VMEM((nch, 16), jnp.float32),
        ],
    )
    flat = run(params)
    return tuple(
        f.reshape(_B, nch, _IMG // s, _IMG // s)
        for f, s in zip(flat, _STRIDES))


# speedup vs baseline: 10.8068x; 1.0019x over previous
import functools
import math

import jax
import jax.numpy as jnp
from jax import lax
from jax.experimental import pallas as pl
from jax.experimental.pallas import tpu as pltpu
from jax.experimental.pallas import tpu_sc as plsc

_STRIDES = (8, 16, 32, 64, 128)
_IMG = 512
_NCLS = 21
_NCH = 4 + 1 + _NCLS
_NB = 64
_B = 8
_NC = 2
_NS = 16
_NW = _NC * _NS
_LANES = 16


def _thresholds():
    result = []
    last = _IMG
    for i in range(len(_STRIDES) - 1, -1, -1):
        s = _STRIDES[i]
        px = float(s) / _IMG
        th_max = math.ceil(last / s)
        if th_max % 2:
            th_max += 1
        th_min = th_max // 2
        last = th_min * s
        if i == 0:
            th_min = 1
        result.append((th_min * px, th_max * px))
    return tuple(result[::-1])

_THS = _thresholds()


def _sqrt_nr(a):
    i = lax.bitcast_convert_type(a, jnp.int32)
    i = jnp.int32(0x5F3759DF) - lax.shift_right_logical(i, 1)
    r = lax.bitcast_convert_type(i, jnp.float32)
    for _ in range(3):
        r = r * (1.5 - 0.5 * a * r * r)
    return a * r


def _sc_mapper(boxes_hbm, labels_hbm, o0, o1, o2, o3, o4,
               boxes_v, labels_v, areas_v, stage0, stage1, stage2, stage3,
               stage4):
    wid = lax.axis_index("s") * _NC + lax.axis_index("c")

    pltpu.sync_copy(boxes_hbm, boxes_v)
    pltpu.sync_copy(labels_hbm, labels_v)

    iota = lax.iota(jnp.int32, _LANES)
    iota4 = iota * 4
    zeros = jnp.zeros((_LANES,), jnp.float32)

    def do_chunk(img, base, lev, store):
        s = _STRIDES[lev]
        size = _IMG // s
        log2s = size.bit_length() - 1
        th0, th1 = _THS[lev]
        scale = float(s) / _IMG

        p = base + iota
        px = jnp.bitwise_and(p, size - 1)
        py = lax.shift_right_logical(p, log2s)
        cx = (px.astype(jnp.float32) + 0.5) * scale
        cy = (py.astype(jnp.float32) + 0.5) * scale

        bbase = img * (_NB * 4)

        def body(i, carry):
            mnmax, warea, widxf = carry
            q = bbase + i * 4
            qv = jnp.full((_LANES,), q, jnp.int32)
            x1 = plsc.load_gather(boxes_v, [qv])
            y1 = plsc.load_gather(boxes_v, [qv + 1])
            x2 = plsc.load_gather(boxes_v, [qv + 2])
            y2 = plsc.load_gather(boxes_v, [qv + 3])
            ar = plsc.load_gather(areas_v, [jnp.full((_LANES,), i, jnp.int32)])
            l = cx - x1
            t = cy - y1
            r = x2 - cx
            b = y2 - cy
            mn = jnp.minimum(jnp.minimum(l, r), jnp.minimum(t, b))
            mx = jnp.maximum(jnp.maximum(l, r), jnp.maximum(t, b))
            mnmax = jnp.maximum(mnmax, mn)
            pred = (mn >= 0.0) & (mx > th0) & (mx <= th1)
            better = pred & (ar <= warea)
            warea = jnp.where(better, ar, warea)
            widxf = jnp.where(better, i.astype(jnp.float32), widxf)
            return mnmax, warea, widxf

        init = (jnp.full((_LANES,), -1.0, jnp.float32),
                jnp.full((_LANES,), 3.0e38, jnp.float32),
                jnp.full((_LANES,), -1.0, jnp.float32))
        mnmax, _, widxf = lax.fori_loop(0, _NB, body, init, unroll=4)

        anyfg = mnmax >= 0.0
        haswin = widxf >= 0.0
        wi = jnp.where(haswin, widxf, 0.0).astype(jnp.int32)
        wq = wi * 4 + jnp.full((_LANES,), bbase, jnp.int32)
        x1w = plsc.load_gather(boxes_v, [wq])
        y1w = plsc.load_gather(boxes_v, [wq + 1])
        x2w = plsc.load_gather(boxes_v, [wq + 2])
        y2w = plsc.load_gather(boxes_v, [wq + 3])
        labw = plsc.load_gather(labels_v, [wi + jnp.full((_LANES,), img * _NB,
                                                         jnp.int32)])
        lr = cx - x1w
        tr = cy - y1w
        rr = x2w - cx
        br = y2w - cy
        arg = ((jnp.minimum(lr, rr) / jnp.maximum(lr, rr)) *
               (jnp.minimum(tr, br) / jnp.maximum(tr, br)))
        pos = haswin & (arg > 0.0)
        safe = jnp.where(pos, arg, 1.0)
        cen = jnp.where(pos, _sqrt_nr(safe), 0.0)
        labm = jnp.where(haswin, labw, 0)

        store(0, jnp.where(haswin, lr, 0.0))
        store(1, jnp.where(haswin, tr, 0.0))
        store(2, jnp.where(haswin, rr, 0.0))
        store(3, jnp.where(haswin, br, 0.0))
        store(4, cen)
        store(5, jnp.where(anyfg, 0.0, 1.0))
        for ch in range(1, _NCLS):
            store(5 + ch, jnp.where(labm == ch, 1.0, 0.0))

    def img_body(img, _):
        for k in range(_NB // _LANES):
            qb = img * (_NB * 4) + k * _LANES * 4
            qv = iota4 + jnp.full((_LANES,), qb, jnp.int32)
            x1v = plsc.load_gather(boxes_v, [qv])
            y1v = plsc.load_gather(boxes_v, [qv + 1])
            x2v = plsc.load_gather(boxes_v, [qv + 2])
            y2v = plsc.load_gather(boxes_v, [qv + 3])
            areas_v[pl.ds(k * _LANES, _LANES)] = (x2v - x1v) * (y2v - y1v)

        b0 = jnp.remainder(wid + 24 * img, 32)

        @pl.when(b0 < 8)
        def _():
            def c0(c, carry):
                row = lax.shift_right_logical(c, 2)
                col = jnp.bitwise_and(c, 3) * _LANES

                def st(ch, v):
                    stage0[ch, row, pl.ds(col, _LANES)] = v
                do_chunk(img, b0 * 512 + c * _LANES, 0, st)
                return carry
            lax.fori_loop(0, 32, c0, 0)
            pltpu.sync_copy(stage0, o0.at[img, :, pl.ds(b0 * 8, 8), :])

        b1 = jnp.remainder(wid + 28 * img, 32)

        @pl.when(b1 < 4)
        def _():
            def c1(c, carry):
                row = lax.shift_right_logical(c, 1)
                col = jnp.bitwise_and(c, 1) * _LANES

                def st(ch, v):
                    stage1[ch, row, pl.ds(col, _LANES)] = v
                do_chunk(img, b1 * 256 + c * _LANES, 1, st)
                return carry
            lax.fori_loop(0, 16, c1, 0)
            pltpu.sync_copy(stage1, o1.at[img, :, pl.ds(b1 * 8, 8), :])

        b2 = jnp.remainder(wid + 30 * img, 32)

        @pl.when(b2 < 2)
        def _():
            def c2(c, carry):
                def st(ch, v):
                    stage2[ch, c, pl.ds(0, _LANES)] = v
                do_chunk(img, b2 * 128 + c * _LANES, 2, st)
                return carry
            lax.fori_loop(0, 8, c2, 0)
            pltpu.sync_copy(stage2, o2.at[img, :, pl.ds(b2 * 8, 8), :])

        @pl.when(wid == 16 + img)
        def _():
            def c3(c, carry):
                def st(ch, v):
                    stage3[ch, pl.ds(c * _LANES, _LANES)] = v
                do_chunk(img, c * _LANES, 3, st)
                return carry
            lax.fori_loop(0, 4, c3, 0)
            pltpu.sync_copy(stage3, o3.at[img])

        @pl.when(wid == 24 + img)
        def _():
            def st(ch, v):
                stage4[ch, pl.ds(0, _LANES)] = v
            do_chunk(img, 0, 4, st)
            pltpu.sync_copy(stage4, o4.at[img])

        return 0

    lax.fori_loop(0, _B, img_body, 0)


def kernel(boxes, labels):
    bflat = boxes.reshape(-1)
    lflat = labels.reshape(-1)

    out_type = (
        jax.ShapeDtypeStruct((_B, _NCH, 64, 64), jnp.float32),
        jax.ShapeDtypeStruct((_B, _NCH, 32, 32), jnp.float32),
        jax.ShapeDtypeStruct((_B, _NCH, 16, 16), jnp.float32),
        jax.ShapeDtypeStruct((_B, _NCH, 64), jnp.float32),
        jax.ShapeDtypeStruct((_B, _NCH, 16), jnp.float32),
    )

    mesh = plsc.VectorSubcoreMesh(core_axis_name="c", subcore_axis_name="s",
                                  num_cores=_NC, num_subcores=_NS)
    run = pl.kernel(
        _sc_mapper,
        out_type=out_type,
        mesh=mesh,
        compiler_params=pltpu.CompilerParams(needs_layout_passes=False),
        scratch_types=[
            pltpu.VMEM((_B * _NB * 4,), jnp.float32),
            pltpu.VMEM((_B * _NB,), jnp.int32),
            pltpu.VMEM((_NB,), jnp.float32),
            pltpu.VMEM((_NCH, 8, 64), jnp.float32),
            pltpu.VMEM((_NCH, 8, 32), jnp.float32),
            pltpu.VMEM((_NCH, 8, 16), jnp.float32),
            pltpu.VMEM((_NCH, 64), jnp.float32),
            pltpu.VMEM((_NCH, 16), jnp.float32),
        ],
    )
    o0, o1, o2, o3, o4 = run(bflat, lflat)
    return (o0, o1, o2,
            o3.reshape(_B, _NCH, 8, 8),
            o4.reshape(_B, _NCH, 4, 4))


# speedup vs baseline: 10.9690x; 1.0150x over previous
import functools
import math

import jax
import jax.numpy as jnp
from jax import lax
from jax.experimental import pallas as pl
from jax.experimental.pallas import tpu as pltpu
from jax.experimental.pallas import tpu_sc as plsc

_STRIDES = (8, 16, 32, 64, 128)
_IMG = 512
_NCLS = 21
_NCH = 4 + 1 + _NCLS
_NB = 64
_B = 8
_NC = 2
_NS = 16
_NW = _NC * _NS
_LANES = 16


def _thresholds():
    result = []
    last = _IMG
    for i in range(len(_STRIDES) - 1, -1, -1):
        s = _STRIDES[i]
        px = float(s) / _IMG
        th_max = math.ceil(last / s)
        if th_max % 2:
            th_max += 1
        th_min = th_max // 2
        last = th_min * s
        if i == 0:
            th_min = 1
        result.append((th_min * px, th_max * px))
    return tuple(result[::-1])

_THS = _thresholds()


def _sqrt_nr(a):
    i = lax.bitcast_convert_type(a, jnp.int32)
    i = jnp.int32(0x5F3759DF) - lax.shift_right_logical(i, 1)
    r = lax.bitcast_convert_type(i, jnp.float32)
    for _ in range(3):
        r = r * (1.5 - 0.5 * a * r * r)
    return a * r


def _sc_mapper(boxes_hbm, labels_hbm, o0, o1, o2, o3, o4,
               boxes_v, labels_v, areas_v, stage0, stage1, stage2, stage3,
               stage4):
    wid = lax.axis_index("s") * _NC + lax.axis_index("c")

    pltpu.sync_copy(boxes_hbm, boxes_v)
    pltpu.sync_copy(labels_hbm, labels_v)

    iota = lax.iota(jnp.int32, _LANES)
    iota4 = iota * 4
    zeros = jnp.zeros((_LANES,), jnp.float32)

    def do_chunk(img, base, lev, store):
        s = _STRIDES[lev]
        size = _IMG // s
        log2s = size.bit_length() - 1
        th0, th1 = _THS[lev]
        scale = float(s) / _IMG

        p = base + iota
        px = jnp.bitwise_and(p, size - 1)
        py = lax.shift_right_logical(p, log2s)
        cx = (px.astype(jnp.float32) + 0.5) * scale
        cy = (py.astype(jnp.float32) + 0.5) * scale

        bbase = img * (_NB * 4)

        def body(i, carry):
            mnmax, warea, widxf = carry
            qv = jnp.full((_LANES,), bbase + i * 4, jnp.int32)
            x1 = plsc.load_gather(boxes_v, [qv])
            y1 = plsc.load_gather(boxes_v, [qv + 1])
            x2 = plsc.load_gather(boxes_v, [qv + 2])
            y2 = plsc.load_gather(boxes_v, [qv + 3])
            ar = plsc.load_gather(areas_v, [jnp.full((_LANES,), i, jnp.int32)])
            l = cx - x1
            t = cy - y1
            r = x2 - cx
            b = y2 - cy
            mn = jnp.minimum(jnp.minimum(l, r), jnp.minimum(t, b))
            mx = jnp.maximum(jnp.maximum(l, r), jnp.maximum(t, b))
            mnmax = jnp.maximum(mnmax, mn)
            pred = (mn >= 0.0) & (mx > th0) & (mx <= th1)
            better = pred & (ar <= warea)
            warea = jnp.where(better, ar, warea)
            widxf = jnp.where(better, i.astype(jnp.float32), widxf)
            return mnmax, warea, widxf

        init = (jnp.full((_LANES,), -1.0, jnp.float32),
                jnp.full((_LANES,), 3.0e38, jnp.float32),
                jnp.full((_LANES,), -1.0, jnp.float32))
        mnmax, _, widxf = lax.fori_loop(0, _NB, body, init)

        anyfg = mnmax >= 0.0
        haswin = widxf >= 0.0
        wi = jnp.where(haswin, widxf, 0.0).astype(jnp.int32)
        wq = wi * 4 + jnp.full((_LANES,), bbase, jnp.int32)
        x1w = plsc.load_gather(boxes_v, [wq])
        y1w = plsc.load_gather(boxes_v, [wq + 1])
        x2w = plsc.load_gather(boxes_v, [wq + 2])
        y2w = plsc.load_gather(boxes_v, [wq + 3])
        labw = plsc.load_gather(labels_v, [wi + jnp.full((_LANES,), img * _NB,
                                                         jnp.int32)])
        lr = cx - x1w
        tr = cy - y1w
        rr = x2w - cx
        br = y2w - cy
        arg = ((jnp.minimum(lr, rr) / jnp.maximum(lr, rr)) *
               (jnp.minimum(tr, br) / jnp.maximum(tr, br)))
        pos = haswin & (arg > 0.0)
        safe = jnp.where(pos, arg, 1.0)
        cen = jnp.where(pos, _sqrt_nr(safe), 0.0)
        labm = jnp.where(haswin, labw, 0)

        store(0, jnp.where(haswin, lr, 0.0))
        store(1, jnp.where(haswin, tr, 0.0))
        store(2, jnp.where(haswin, rr, 0.0))
        store(3, jnp.where(haswin, br, 0.0))
        store(4, cen)
        store(5, jnp.where(anyfg, 0.0, 1.0))
        for ch in range(1, _NCLS):
            store(5 + ch, jnp.where(labm == ch, 1.0, 0.0))

    def img_body(img, _):
        for k in range(_NB // _LANES):
            qb = img * (_NB * 4) + k * _LANES * 4
            qv = iota4 + jnp.full((_LANES,), qb, jnp.int32)
            x1v = plsc.load_gather(boxes_v, [qv])
            y1v = plsc.load_gather(boxes_v, [qv + 1])
            x2v = plsc.load_gather(boxes_v, [qv + 2])
            y2v = plsc.load_gather(boxes_v, [qv + 3])
            areas_v[pl.ds(k * _LANES, _LANES)] = (x2v - x1v) * (y2v - y1v)


        b0 = jnp.remainder(wid + 24 * img, 32)

        @pl.when(b0 < 8)
        def _():
            def c0(c, carry):
                row = lax.shift_right_logical(c, 2)
                col = jnp.bitwise_and(c, 3) * _LANES

                def st(ch, v):
                    stage0[ch, row, pl.ds(col, _LANES)] = v
                do_chunk(img, b0 * 512 + c * _LANES, 0, st)
                return carry
            lax.fori_loop(0, 32, c0, 0)
            pltpu.sync_copy(stage0, o0.at[img, :, pl.ds(b0 * 8, 8), :])

        b1 = jnp.remainder(wid + 28 * img, 32)

        @pl.when(b1 < 4)
        def _():
            def c1(c, carry):
                row = lax.shift_right_logical(c, 1)
                col = jnp.bitwise_and(c, 1) * _LANES

                def st(ch, v):
                    stage1[ch, row, pl.ds(col, _LANES)] = v
                do_chunk(img, b1 * 256 + c * _LANES, 1, st)
                return carry
            lax.fori_loop(0, 16, c1, 0)
            pltpu.sync_copy(stage1, o1.at[img, :, pl.ds(b1 * 8, 8), :])

        b2 = jnp.remainder(wid + 30 * img, 32)

        @pl.when(b2 < 2)
        def _():
            def c2(c, carry):
                def st(ch, v):
                    stage2[ch, c, pl.ds(0, _LANES)] = v
                do_chunk(img, b2 * 128 + c * _LANES, 2, st)
                return carry
            lax.fori_loop(0, 8, c2, 0)
            pltpu.sync_copy(stage2, o2.at[img, :, pl.ds(b2 * 8, 8), :])

        @pl.when(wid == 16 + img)
        def _():
            def c3(c, carry):
                def st(ch, v):
                    stage3[ch, pl.ds(c * _LANES, _LANES)] = v
                do_chunk(img, c * _LANES, 3, st)
                return carry
            lax.fori_loop(0, 4, c3, 0)
            pltpu.sync_copy(stage3, o3.at[img])

        @pl.when(wid == 24 + img)
        def _():
            def st(ch, v):
                stage4[ch, pl.ds(0, _LANES)] = v
            do_chunk(img, 0, 4, st)
            pltpu.sync_copy(stage4, o4.at[img])

        return 0

    lax.fori_loop(0, _B, img_body, 0)


def kernel(boxes, labels):
    bflat = boxes.reshape(-1)
    lflat = labels.reshape(-1)

    out_type = (
        jax.ShapeDtypeStruct((_B, _NCH, 64, 64), jnp.float32),
        jax.ShapeDtypeStruct((_B, _NCH, 32, 32), jnp.float32),
        jax.ShapeDtypeStruct((_B, _NCH, 16, 16), jnp.float32),
        jax.ShapeDtypeStruct((_B, _NCH, 64), jnp.float32),
        jax.ShapeDtypeStruct((_B, _NCH, 16), jnp.float32),
    )

    mesh = plsc.VectorSubcoreMesh(core_axis_name="c", subcore_axis_name="s",
                                  num_cores=_NC, num_subcores=_NS)
    run = pl.kernel(
        _sc_mapper,
        out_type=out_type,
        mesh=mesh,
        compiler_params=pltpu.CompilerParams(needs_layout_passes=False),
        scratch_types=[
            pltpu.VMEM((_B * _NB * 4,), jnp.float32),
            pltpu.VMEM((_B * _NB,), jnp.int32),
            pltpu.VMEM((_NB,), jnp.float32),
            pltpu.VMEM((_NCH, 8, 64), jnp.float32),
            pltpu.VMEM((_NCH, 8, 32), jnp.float32),
            pltpu.VMEM((_NCH, 8, 16), jnp.float32),
            pltpu.VMEM((_NCH, 64), jnp.float32),
            pltpu.VMEM((_NCH, 16), jnp.float32),
        ],
    )
    o0, o1, o2, o3, o4 = run(bflat, lflat)
    return (o0, o1, o2,
            o3.reshape(_B, _NCH, 8, 8),
            o4.reshape(_B, _NCH, 4, 4))


# speedup vs baseline: 13.1077x; 1.1950x over previous
import functools
import math

import jax
import jax.numpy as jnp
from jax import lax
from jax.experimental import pallas as pl
from jax.experimental.pallas import tpu as pltpu
from jax.experimental.pallas import tpu_sc as plsc

_STRIDES = (8, 16, 32, 64, 128)
_IMG = 512
_NCLS = 21
_NCH = 4 + 1 + _NCLS
_NB = 64
_B = 8
_NC = 2
_NS = 16
_LANES = 16


def _thresholds():
    result = []
    last = _IMG
    for i in range(len(_STRIDES) - 1, -1, -1):
        s = _STRIDES[i]
        px = float(s) / _IMG
        th_max = math.ceil(last / s)
        if th_max % 2:
            th_max += 1
        th_min = th_max // 2
        last = th_min * s
        if i == 0:
            th_min = 1
        result.append((th_min * px, th_max * px))
    return tuple(result[::-1])

_THS = _thresholds()


def _sqrt_nr(a):
    i = lax.bitcast_convert_type(a, jnp.int32)
    i = jnp.int32(0x5F3759DF) - lax.shift_right_logical(i, 1)
    r = lax.bitcast_convert_type(i, jnp.float32)
    for _ in range(3):
        r = r * (1.5 - 0.5 * a * r * r)
    return a * r


def _sc_mapper(boxes_hbm, labels_hbm, o0, o1, o2, o3, o4,
               boxes_v, labels_v, areas_v, stage0, stage1, stage2, stage3,
               stage4, win_v, mn_v):
    wid = lax.axis_index("s") * _NC + lax.axis_index("c")

    pltpu.sync_copy(boxes_hbm, boxes_v)
    pltpu.sync_copy(labels_hbm, labels_v)

    iota = lax.iota(jnp.int32, _LANES)
    iota4 = iota * 4

    def pix_coords(lev, p):
        size = _IMG // _STRIDES[lev]
        log2s = size.bit_length() - 1
        scale = float(_STRIDES[lev]) / _IMG
        px = jnp.bitwise_and(p, size - 1)
        py = lax.shift_right_logical(p, log2s)
        cx = (px.astype(jnp.float32) + 0.5) * scale
        cy = (py.astype(jnp.float32) + 0.5) * scale
        return cx, cy

    def scan_block(img, base, lev, nchunks, woff):
        size = _IMG // _STRIDES[lev]
        th0, th1 = _THS[lev]
        cpr = size // _LANES

        cxs, cys = [], []
        for j in range(nchunks):
            cx, cy = pix_coords(lev, base + j * _LANES + iota)
            cxs.append(cx)
            cys.append(cy)

        bbase = img * (_NB * 4)

        def body(i, carry):
            mnm = list(carry[0])
            war = list(carry[1])
            wix = list(carry[2])
            qv = jnp.full((_LANES,), bbase + i * 4, jnp.int32)
            x1 = plsc.load_gather(boxes_v, [qv])
            y1 = plsc.load_gather(boxes_v, [qv + 1])
            x2 = plsc.load_gather(boxes_v, [qv + 2])
            y2 = plsc.load_gather(boxes_v, [qv + 3])
            ar = plsc.load_gather(areas_v, [jnp.full((_LANES,), i, jnp.int32)])
            fi = i.astype(jnp.float32)
            tbs = []
            if cpr >= 1:
                for r in range(nchunks // cpr):
                    cy = cys[r * cpr]
                    t = cy - y1
                    b = y2 - cy
                    tbs.append((jnp.minimum(t, b), jnp.maximum(t, b)))
            for j in range(nchunks):
                if cpr >= 1:
                    mint, maxt = tbs[j // cpr]
                else:
                    t = cys[j] - y1
                    b = y2 - cys[j]
                    mint = jnp.minimum(t, b)
                    maxt = jnp.maximum(t, b)
                l = cxs[j] - x1
                rr = x2 - cxs[j]
                mn = jnp.minimum(jnp.minimum(l, rr), mint)
                mx = jnp.maximum(jnp.maximum(l, rr), maxt)
                mnm[j] = jnp.maximum(mnm[j], mn)
                pred = (mn >= 0.0) & (mx > th0) & (mx <= th1)
                better = pred & (ar <= war[j])
                war[j] = jnp.where(better, ar, war[j])
                wix[j] = jnp.where(better, fi, wix[j])
            return (tuple(mnm), tuple(war), tuple(wix))

        neg1 = jnp.full((_LANES,), -1.0, jnp.float32)
        big = jnp.full((_LANES,), 3.0e38, jnp.float32)
        init = (tuple(neg1 for _ in range(nchunks)),
                tuple(big for _ in range(nchunks)),
                tuple(neg1 for _ in range(nchunks)))
        mnm, _, wix = lax.fori_loop(0, _NB, body, init)
        for j in range(nchunks):
            d = pl.ds((woff + j) * _LANES, _LANES)
            win_v[d] = wix[j]
            mn_v[d] = mnm[j]

    def emit_chunks(img, band, lev, nchunks, store):
        bbase = img * (_NB * 4)

        def ebody(c, carry):
            d = pl.ds(c * _LANES, _LANES)
            widxf = win_v[d]
            mnmax = mn_v[d]
            cx, cy = pix_coords(lev, band + c * _LANES + iota)
            anyfg = mnmax >= 0.0
            haswin = widxf >= 0.0
            wi = jnp.where(haswin, widxf, 0.0).astype(jnp.int32)
            wq = wi * 4 + jnp.full((_LANES,), bbase, jnp.int32)
            x1w = plsc.load_gather(boxes_v, [wq])
            y1w = plsc.load_gather(boxes_v, [wq + 1])
            x2w = plsc.load_gather(boxes_v, [wq + 2])
            y2w = plsc.load_gather(boxes_v, [wq + 3])
            labw = plsc.load_gather(
                labels_v, [wi + jnp.full((_LANES,), img * _NB, jnp.int32)])
            lr = cx - x1w
            tr = cy - y1w
            rr = x2w - cx
            br = y2w - cy
            arg = ((jnp.minimum(lr, rr) / jnp.maximum(lr, rr)) *
                   (jnp.minimum(tr, br) / jnp.maximum(tr, br)))
            pos = haswin & (arg > 0.0)
            safe = jnp.where(pos, arg, 1.0)
            cen = jnp.where(pos, _sqrt_nr(safe), 0.0)
            labm = jnp.where(haswin, labw, 0)

            store(0, c, jnp.where(haswin, lr, 0.0))
            store(1, c, jnp.where(haswin, tr, 0.0))
            store(2, c, jnp.where(haswin, rr, 0.0))
            store(3, c, jnp.where(haswin, br, 0.0))
            store(4, c, cen)
            store(5, c, jnp.where(anyfg, 0.0, 1.0))
            for ch in range(1, _NCLS):
                store(5 + ch, c, jnp.where(labm == ch, 1.0, 0.0))
            return carry

        lax.fori_loop(0, nchunks, ebody, 0)

    def img_body(img, _):
        for k in range(_NB // _LANES):
            qb = img * (_NB * 4) + k * _LANES * 4
            qv = iota4 + jnp.full((_LANES,), qb, jnp.int32)
            x1v = plsc.load_gather(boxes_v, [qv])
            y1v = plsc.load_gather(boxes_v, [qv + 1])
            x2v = plsc.load_gather(boxes_v, [qv + 2])
            y2v = plsc.load_gather(boxes_v, [qv + 3])
            areas_v[pl.ds(k * _LANES, _LANES)] = (x2v - x1v) * (y2v - y1v)

        b0 = jnp.remainder(wid + 24 * img, 32)

        @pl.when(b0 < 8)
        def _():
            band = b0 * 512

            def blk(kb, carry):
                scan_block(img, band + kb * 128, 0, 8, kb * 8)
                return carry
            lax.fori_loop(0, 4, blk, 0)

            def st0(ch, c, v):
                stage0[ch, lax.shift_right_logical(c, 2),
                       pl.ds(jnp.bitwise_and(c, 3) * _LANES, _LANES)] = v
            emit_chunks(img, band, 0, 32, st0)
            pltpu.sync_copy(stage0, o0.at[img, :, pl.ds(b0 * 8, 8), :])

        b1 = jnp.remainder(wid + 28 * img, 32)

        @pl.when(b1 < 4)
        def _():
            band = b1 * 256

            def blk(kb, carry):
                scan_block(img, band + kb * 128, 1, 8, kb * 8)
                return carry
            lax.fori_loop(0, 2, blk, 0)

            def st1(ch, c, v):
                stage1[ch, lax.shift_right_logical(c, 1),
                       pl.ds(jnp.bitwise_and(c, 1) * _LANES, _LANES)] = v
            emit_chunks(img, band, 1, 16, st1)
            pltpu.sync_copy(stage1, o1.at[img, :, pl.ds(b1 * 8, 8), :])

        b2 = jnp.remainder(wid + 30 * img, 32)

        @pl.when(b2 < 2)
        def _():
            band = b2 * 128
            scan_block(img, band, 2, 8, 0)

            def st2(ch, c, v):
                stage2[ch, c, pl.ds(0, _LANES)] = v
            emit_chunks(img, band, 2, 8, st2)
            pltpu.sync_copy(stage2, o2.at[img, :, pl.ds(b2 * 8, 8), :])

        @pl.when(wid == 16 + img)
        def _():
            scan_block(img, 0, 3, 4, 0)

            def st3(ch, c, v):
                stage3[ch, pl.ds(c * _LANES, _LANES)] = v
            emit_chunks(img, 0, 3, 4, st3)
            pltpu.sync_copy(stage3, o3.at[img])

        @pl.when(wid == 24 + img)
        def _():
            scan_block(img, 0, 4, 1, 0)

            def st4(ch, c, v):
                stage4[ch, pl.ds(0, _LANES)] = v
            emit_chunks(img, 0, 4, 1, st4)
            pltpu.sync_copy(stage4, o4.at[img])

        return 0

    lax.fori_loop(0, _B, img_body, 0)


def kernel(boxes, labels):
    bflat = boxes.reshape(-1)
    lflat = labels.reshape(-1)

    out_type = (
        jax.ShapeDtypeStruct((_B, _NCH, 64, 64), jnp.float32),
        jax.ShapeDtypeStruct((_B, _NCH, 32, 32), jnp.float32),
        jax.ShapeDtypeStruct((_B, _NCH, 16, 16), jnp.float32),
        jax.ShapeDtypeStruct((_B, _NCH, 64), jnp.float32),
        jax.ShapeDtypeStruct((_B, _NCH, 16), jnp.float32),
    )

    mesh = plsc.VectorSubcoreMesh(core_axis_name="c", subcore_axis_name="s",
                                  num_cores=_NC, num_subcores=_NS)
    run = pl.kernel(
        _sc_mapper,
        out_type=out_type,
        mesh=mesh,
        compiler_params=pltpu.CompilerParams(needs_layout_passes=False),
        scratch_types=[
            pltpu.VMEM((_B * _NB * 4,), jnp.float32),
            pltpu.VMEM((_B * _NB,), jnp.int32),
            pltpu.VMEM((_NB,), jnp.float32),
            pltpu.VMEM((_NCH, 8, 64), jnp.float32),
            pltpu.VMEM((_NCH, 8, 32), jnp.float32),
            pltpu.VMEM((_NCH, 8, 16), jnp.float32),
            pltpu.VMEM((_NCH, 64), jnp.float32),
            pltpu.VMEM((_NCH, 16), jnp.float32),
            pltpu.VMEM((512,), jnp.float32),
            pltpu.VMEM((512,), jnp.float32),
        ],
    )
    o0, o1, o2, o3, o4 = run(bflat, lflat)
    return (o0, o1, o2,
            o3.reshape(_B, _NCH, 8, 8),
            o4.reshape(_B, _NCH, 4, 4))
